# baseline (speedup 1.0000x reference)
import functools

import jax
import jax.numpy as jnp
from jax.experimental import pallas as pl
from jax.experimental.pallas import tpu as pltpu

H = 16
SPARSITY = 0.9
K_CAP = 1024
BISECT_ITERS = 18


def _bf16_dot(a_bf, b_bf, contract_b=0):
    return jax.lax.dot_general(
        a_bf, b_bf, (((1,), (contract_b,)), ((), ())),
        preferred_element_type=jnp.float32)


def _fused_kernel(x_ref, wq_ref, bq_ref, wk_ref, bk_ref, wv_ref, bv_ref,
                  wo_ref, bo_ref, o_ref, xbf_scr, k_scr, v_scr,
                  *, k_keep, scale, bq_size):
    h = pl.program_id(1)
    i = pl.program_id(2)

    @pl.when(jnp.logical_and(h == 0, i == 0))
    def _():
        xbf_scr[...] = x_ref[0].astype(jnp.bfloat16)

    @pl.when(i == 0)
    def _():
        kf = _bf16_dot(xbf_scr[...], wk_ref[0].astype(jnp.bfloat16))
        k_scr[...] = (kf + bk_ref[0]).astype(jnp.bfloat16)
        vf = _bf16_dot(xbf_scr[...], wv_ref[0].astype(jnp.bfloat16))
        v_scr[...] = (vf + bv_ref[0]).astype(jnp.bfloat16)

    row0 = i * bq_size
    xq = xbf_scr[pl.ds(row0, bq_size), :]
    qf = _bf16_dot(xq, wq_ref[0].astype(jnp.bfloat16)) + bq_ref[0]
    qb = qf.astype(jnp.bfloat16)

    scores = _bf16_dot(qb, k_scr[...], contract_b=1) * scale

    m = jnp.max(scores, axis=1, keepdims=True)
    lo = jnp.min(scores, axis=1, keepdims=True)
    hi = m
    for _ in range(BISECT_ITERS):
        mid = 0.5 * (lo + hi)
        cnt = jnp.sum((scores >= mid).astype(jnp.float32), axis=1,
                      keepdims=True)
        ge = cnt >= k_keep
        lo = jnp.where(ge, mid, lo)
        hi = jnp.where(ge, hi, mid)

    p = jnp.where(scores >= lo, jnp.exp(scores - m), 0.0)
    l = jnp.sum(p, axis=1, keepdims=True)
    attn = _bf16_dot(p.astype(jnp.bfloat16), v_scr[...]) / l

    contrib = _bf16_dot(attn.astype(jnp.bfloat16),
                        wo_ref[0].astype(jnp.bfloat16))

    @pl.when(h == 0)
    def _():
        o_ref[0, pl.ds(row0, bq_size), :] = contrib + bo_ref[...]

    @pl.when(h != 0)
    def _():
        o_ref[0, pl.ds(row0, bq_size), :] += contrib


def kernel(x, Wq, bq, Wk, bk, Wv, bv, Wo, bo):
    B, N, DIM = x.shape
    DH = DIM // H
    k_keep = min(max(1, int(N * (1.0 - SPARSITY))), K_CAP)
    scale = 1.0 / (DH ** 0.5)
    BQ = min(256, N)
    NBQ = N // BQ

    wq_h = Wq.reshape(DIM, H, DH).transpose(1, 0, 2)
    wk_h = Wk.reshape(DIM, H, DH).transpose(1, 0, 2)
    wv_h = Wv.reshape(DIM, H, DH).transpose(1, 0, 2)
    wo_h = Wo.reshape(H, DH, DIM)
    bq_h = bq.reshape(H, 1, DH)
    bk_h = bk.reshape(H, 1, DH)
    bv_h = bv.reshape(H, 1, DH)
    bo_2 = bo.reshape(1, DIM)

    out = pl.pallas_call(
        functools.partial(_fused_kernel, k_keep=k_keep, scale=scale,
                          bq_size=BQ),
        grid=(B, H, NBQ),
        in_specs=[
            pl.BlockSpec((1, N, DIM), lambda b, h, i: (b, 0, 0)),
            pl.BlockSpec((1, DIM, DH), lambda b, h, i: (h, 0, 0)),
            pl.BlockSpec((1, 1, DH), lambda b, h, i: (h, 0, 0)),
            pl.BlockSpec((1, DIM, DH), lambda b, h, i: (h, 0, 0)),
            pl.BlockSpec((1, 1, DH), lambda b, h, i: (h, 0, 0)),
            pl.BlockSpec((1, DIM, DH), lambda b, h, i: (h, 0, 0)),
            pl.BlockSpec((1, 1, DH), lambda b, h, i: (h, 0, 0)),
            pl.BlockSpec((1, DH, DIM), lambda b, h, i: (h, 0, 0)),
            pl.BlockSpec((1, DIM), lambda b, h, i: (0, 0)),
        ],
        out_specs=pl.BlockSpec((1, N, DIM), lambda b, h, i: (b, 0, 0)),
        out_shape=jax.ShapeDtypeStruct((B, N, DIM), jnp.float32),
        scratch_shapes=[
            pltpu.VMEM((N, DIM), jnp.bfloat16),
            pltpu.VMEM((N, DH), jnp.bfloat16),
            pltpu.VMEM((N, DH), jnp.bfloat16),
        ],
        compiler_params=pltpu.CompilerParams(
            dimension_semantics=("arbitrary", "arbitrary", "arbitrary")),
    )(x, wq_h, bq_h, wk_h, bk_h, wv_h, bv_h, wo_h, bo_2)

    return out


# speedup vs baseline: 1.2103x; 1.2103x over previous
import functools

import jax
import jax.numpy as jnp
from jax.experimental import pallas as pl
from jax.experimental.pallas import tpu as pltpu

H = 16
SPARSITY = 0.9
K_CAP = 1024
BISECT_ITERS = 16


def _bdot(a, b):
    return jax.lax.dot_general(
        a.astype(jnp.bfloat16), b.astype(jnp.bfloat16),
        (((1,), (0,)), ((), ())),
        preferred_element_type=jnp.float32)


def _proj_kernel(x_ref, wq_ref, bq_ref, wk_ref, bk_ref, wv_ref, bv_ref,
                 q_ref, k_ref, v_ref, *, dh):
    x = x_ref[0]
    bn = x.shape[0]
    def split(t):
        return t.reshape(bn, H, dh).transpose(1, 0, 2)
    q_ref[...] = split(_bdot(x, wq_ref[...]) + bq_ref[...])
    k_ref[...] = split(_bdot(x, wk_ref[...]) + bk_ref[...])
    v_ref[...] = split(_bdot(x, wv_ref[...]) + bv_ref[...])


def _out_proj_kernel(a_ref, wo_ref, bo_ref, o_ref):
    a = a_ref[...]
    h, bn, dh = a.shape
    merged = a.transpose(1, 0, 2).reshape(bn, h * dh)
    o_ref[0] = _bdot(merged, wo_ref[...]) + bo_ref[...]


def _attn_kernel(q_ref, k_ref, v_ref, o_ref, *, k_keep, scale):
    q = q_ref[0]
    k = k_ref[0]
    v = v_ref[0]
    scores = jax.lax.dot_general(
        (q * scale).astype(jnp.bfloat16), k.astype(jnp.bfloat16),
        (((1,), (1,)), ((), ())),
        preferred_element_type=jnp.float32)

    m = jnp.max(scores, axis=1, keepdims=True)
    lo = jnp.min(scores, axis=1, keepdims=True)
    hi = m
    for _ in range(BISECT_ITERS):
        mid = 0.5 * (lo + hi)
        cnt = jnp.sum((scores >= mid).astype(jnp.float32), axis=1,
                      keepdims=True)
        ge = cnt >= k_keep
        lo = jnp.where(ge, mid, lo)
        hi = jnp.where(ge, hi, mid)

    p = jnp.where(scores >= lo, jnp.exp(scores - m), 0.0)
    l = jnp.sum(p, axis=1, keepdims=True)
    out = _bdot(p, v)
    o_ref[0] = out / l


def kernel(x, Wq, bq, Wk, bk, Wv, bv, Wo, bo):
    B, N, DIM = x.shape
    DH = DIM // H
    BH = B * H
    k_keep = min(max(1, int(N * (1.0 - SPARSITY))), K_CAP)
    scale = 1.0 / (DH ** 0.5)

    BR = 256
    NB = N // BR
    bq2, bk2, bv2, bo2 = (b.reshape(1, DIM) for b in (bq, bk, bv, bo))

    q, k, v = pl.pallas_call(
        functools.partial(_proj_kernel, dh=DH),
        grid=(B, NB),
        in_specs=[
            pl.BlockSpec((1, BR, DIM), lambda b, i: (b, i, 0)),
            pl.BlockSpec((DIM, DIM), lambda b, i: (0, 0)),
            pl.BlockSpec((1, DIM), lambda b, i: (0, 0)),
            pl.BlockSpec((DIM, DIM), lambda b, i: (0, 0)),
            pl.BlockSpec((1, DIM), lambda b, i: (0, 0)),
            pl.BlockSpec((DIM, DIM), lambda b, i: (0, 0)),
            pl.BlockSpec((1, DIM), lambda b, i: (0, 0)),
        ],
        out_specs=[
            pl.BlockSpec((H, BR, DH), lambda b, i: (b, i, 0)),
            pl.BlockSpec((H, BR, DH), lambda b, i: (b, i, 0)),
            pl.BlockSpec((H, BR, DH), lambda b, i: (b, i, 0)),
        ],
        out_shape=[jax.ShapeDtypeStruct((BH, N, DH), jnp.float32)] * 3,
        compiler_params=pltpu.CompilerParams(
            dimension_semantics=("arbitrary", "arbitrary")),
    )(x, Wq, bq2, Wk, bk2, Wv, bv2)

    BQ = min(256, N)
    attn = pl.pallas_call(
        functools.partial(_attn_kernel, k_keep=k_keep, scale=scale),
        grid=(BH, N // BQ),
        in_specs=[
            pl.BlockSpec((1, BQ, DH), lambda h, i: (h, i, 0)),
            pl.BlockSpec((1, N, DH), lambda h, i: (h, 0, 0)),
            pl.BlockSpec((1, N, DH), lambda h, i: (h, 0, 0)),
        ],
        out_specs=pl.BlockSpec((1, BQ, DH), lambda h, i: (h, i, 0)),
        out_shape=jax.ShapeDtypeStruct((BH, N, DH), jnp.float32),
        compiler_params=pltpu.CompilerParams(
            dimension_semantics=("parallel", "arbitrary")),
    )(q, k, v)

    out = pl.pallas_call(
        _out_proj_kernel,
        grid=(B, NB),
        in_specs=[
            pl.BlockSpec((H, BR, DH), lambda b, i: (b, i, 0)),
            pl.BlockSpec((DIM, DIM), lambda b, i: (0, 0)),
            pl.BlockSpec((1, DIM), lambda b, i: (0, 0)),
        ],
        out_specs=pl.BlockSpec((1, BR, DIM), lambda b, i: (b, i, 0)),
        out_shape=jax.ShapeDtypeStruct((B, N, DIM), jnp.float32),
        compiler_params=pltpu.CompilerParams(
            dimension_semantics=("arbitrary", "arbitrary")),
    )(attn, Wo, bo2)

    return out


# speedup vs baseline: 1.2138x; 1.0029x over previous
import functools

import jax
import jax.numpy as jnp
from jax.experimental import pallas as pl
from jax.experimental.pallas import tpu as pltpu

H = 16
SPARSITY = 0.9
K_CAP = 1024
BISECT_ITERS = 16


def _bdot(a, b):
    return jax.lax.dot_general(
        a.astype(jnp.bfloat16), b.astype(jnp.bfloat16),
        (((1,), (0,)), ((), ())),
        preferred_element_type=jnp.float32)


def _proj_kernel(x_ref, wq_ref, bq_ref, wk_ref, bk_ref, wv_ref, bv_ref,
                 q_ref, k_ref, v_ref, *, dh, scale):
    x = x_ref[0]
    bn = x.shape[0]
    def split(t):
        return t.astype(jnp.bfloat16).reshape(bn, H, dh).transpose(1, 0, 2)
    q_ref[...] = split((_bdot(x, wq_ref[...]) + bq_ref[...]) * scale)
    k_ref[...] = split(_bdot(x, wk_ref[...]) + bk_ref[...])
    v_ref[...] = split(_bdot(x, wv_ref[...]) + bv_ref[...])


def _out_proj_kernel(a_ref, wo_ref, bo_ref, o_ref):
    a = a_ref[...]
    h, bn, dh = a.shape
    merged = a.transpose(1, 0, 2).reshape(bn, h * dh)
    o_ref[0] = jax.lax.dot_general(
        merged, wo_ref[...].astype(jnp.bfloat16), (((1,), (0,)), ((), ())),
        preferred_element_type=jnp.float32) + bo_ref[...]


def _attn_kernel(q_ref, k_ref, v_ref, o_ref, *, k_keep):
    q = q_ref[0]
    k = k_ref[0]
    v = v_ref[0]
    scores = jax.lax.dot_general(
        q, k, (((1,), (1,)), ((), ())),
        preferred_element_type=jnp.float32)

    m = jnp.max(scores, axis=1, keepdims=True)
    lo = jnp.min(scores, axis=1, keepdims=True)
    hi = m
    for _ in range(BISECT_ITERS):
        mid = 0.5 * (lo + hi)
        cnt = jnp.sum((scores >= mid).astype(jnp.float32), axis=1,
                      keepdims=True)
        ge = cnt >= k_keep
        lo = jnp.where(ge, mid, lo)
        hi = jnp.where(ge, hi, mid)

    p = jnp.where(scores >= lo, jnp.exp(scores - m), 0.0)
    l = jnp.sum(p, axis=1, keepdims=True)
    out = jax.lax.dot_general(
        p.astype(jnp.bfloat16), v, (((1,), (0,)), ((), ())),
        preferred_element_type=jnp.float32)
    o_ref[0] = (out / l).astype(jnp.bfloat16)


def kernel(x, Wq, bq, Wk, bk, Wv, bv, Wo, bo):
    B, N, DIM = x.shape
    DH = DIM // H
    BH = B * H
    k_keep = min(max(1, int(N * (1.0 - SPARSITY))), K_CAP)
    scale = 1.0 / (DH ** 0.5)

    BR = 256
    NB = N // BR
    bq2, bk2, bv2, bo2 = (b.reshape(1, DIM) for b in (bq, bk, bv, bo))

    q, k, v = pl.pallas_call(
        functools.partial(_proj_kernel, dh=DH, scale=scale),
        grid=(B, NB),
        in_specs=[
            pl.BlockSpec((1, BR, DIM), lambda b, i: (b, i, 0)),
            pl.BlockSpec((DIM, DIM), lambda b, i: (0, 0)),
            pl.BlockSpec((1, DIM), lambda b, i: (0, 0)),
            pl.BlockSpec((DIM, DIM), lambda b, i: (0, 0)),
            pl.BlockSpec((1, DIM), lambda b, i: (0, 0)),
            pl.BlockSpec((DIM, DIM), lambda b, i: (0, 0)),
            pl.BlockSpec((1, DIM), lambda b, i: (0, 0)),
        ],
        out_specs=[
            pl.BlockSpec((H, BR, DH), lambda b, i: (b, i, 0)),
            pl.BlockSpec((H, BR, DH), lambda b, i: (b, i, 0)),
            pl.BlockSpec((H, BR, DH), lambda b, i: (b, i, 0)),
        ],
        out_shape=[jax.ShapeDtypeStruct((BH, N, DH), jnp.bfloat16)] * 3,
        compiler_params=pltpu.CompilerParams(
            dimension_semantics=("arbitrary", "arbitrary")),
    )(x, Wq, bq2, Wk, bk2, Wv, bv2)

    BQ = min(256, N)
    attn = pl.pallas_call(
        functools.partial(_attn_kernel, k_keep=k_keep),
        grid=(BH, N // BQ),
        in_specs=[
            pl.BlockSpec((1, BQ, DH), lambda h, i: (h, i, 0)),
            pl.BlockSpec((1, N, DH), lambda h, i: (h, 0, 0)),
            pl.BlockSpec((1, N, DH), lambda h, i: (h, 0, 0)),
        ],
        out_specs=pl.BlockSpec((1, BQ, DH), lambda h, i: (h, i, 0)),
        out_shape=jax.ShapeDtypeStruct((BH, N, DH), jnp.bfloat16),
        compiler_params=pltpu.CompilerParams(
            dimension_semantics=("parallel", "arbitrary")),
    )(q, k, v)

    out = pl.pallas_call(
        _out_proj_kernel,
        grid=(B, NB),
        in_specs=[
            pl.BlockSpec((H, BR, DH), lambda b, i: (b, i, 0)),
            pl.BlockSpec((DIM, DIM), lambda b, i: (0, 0)),
            pl.BlockSpec((1, DIM), lambda b, i: (0, 0)),
        ],
        out_specs=pl.BlockSpec((1, BR, DIM), lambda b, i: (b, i, 0)),
        out_shape=jax.ShapeDtypeStruct((B, N, DIM), jnp.float32),
        compiler_params=pltpu.CompilerParams(
            dimension_semantics=("arbitrary", "arbitrary")),
    )(attn, Wo, bo2)

    return out
